# Initial kernel scaffold; baseline (speedup 1.0000x reference)
import jax
import jax.numpy as jnp
from jax.experimental import pallas as pl


def kernel(x, he_index, he_attr, he_count, node_pos, response_idx, batch, params):
    raise NotImplementedError("write your pallas kernel here")


# speedup vs baseline: 3.2407x; 3.2407x over previous
import functools

import jax
import jax.numpy as jnp
from jax import lax
from jax.experimental import pallas as pl
from jax.experimental.pallas import tpu as pltpu
from jax.experimental.pallas import tpu_sc as plsc

F32 = jnp.float32


def _ln(v, g, b):
    mu = jnp.mean(v, axis=-1, keepdims=True)
    var = jnp.mean((v - mu) ** 2, axis=-1, keepdims=True)
    return (v - mu) / jnp.sqrt(var + 1e-5) * g + b


_NS = 16
_SUP = 8


def _sc_scatter_add(table2, g0, g1, sidx, T, D, split_by_core):
    MR = sidx.shape[0]
    rows_per_tile = MR // (2 * _NS) if split_by_core else MR // _NS
    n_sup = rows_per_tile // _SUP
    ZB = 200
    NB = T // ZB
    nzi = -(-NB // _NS)
    gather = table2 is not None

    def body(*refs):
        if gather:
            (table_ref, g0_ref, g1_ref, sidx_ref, zeros_ref, out_ref,
             gvx, svx, rows, zbuf, acc, sem) = refs
        else:
            (ones_ref, sidx_ref, zeros_ref, out_ref,
             svx, rows, zbuf, acc, sem) = refs
        c = lax.axis_index("c")
        s = lax.axis_index("s")

        if not gather:
            pltpu.sync_copy(ones_ref, rows)

        pltpu.sync_copy(zeros_ref, zbuf)

        def zblk(i, carry):
            b = s + _NS * i

            @pl.when(b < NB)
            def _():
                pltpu.sync_copy(zbuf, acc.at[pl.ds(b * ZB, ZB)])

            return carry

        lax.fori_loop(0, nzi, zblk, 0)
        plsc.subcore_barrier()

        if split_by_core:
            tile_row0 = (c * _NS + s) * rows_per_tile
        else:
            tile_row0 = s * rows_per_tile

        def sup_body(j, carry):
            r0 = tile_row0 + j * _SUP

            if gather:
                @pl.when(c == 0)
                def _():
                    pltpu.sync_copy(g0_ref.at[pl.ds(r0, _SUP)], gvx)

                @pl.when(c == 1)
                def _():
                    pltpu.sync_copy(g1_ref.at[pl.ds(r0, _SUP)], gvx)

            pltpu.sync_copy(sidx_ref.at[pl.ds(r0, _SUP)], svx)
            for b in range(_SUP):
                if gather:
                    pltpu.async_copy(table_ref.at[gvx.at[b]], rows,
                                     sem).wait()
                pltpu.sync_copy(rows, acc.at[svx.at[b]], add=True)
            return carry

        lax.fori_loop(0, n_sup, sup_body, 0)
        plsc.subcore_barrier()

        def wblk(i, carry):
            b = s + _NS * i

            @pl.when(b < NB)
            def _():
                pltpu.sync_copy(acc.at[pl.ds(b * ZB, ZB)],
                                out_ref.at[pl.ds(c * T + b * ZB, ZB)])

            return carry

        lax.fori_loop(0, nzi, wblk, 0)

    mesh = plsc.VectorSubcoreMesh(core_axis_name="c", subcore_axis_name="s")
    if gather:
        scratch = [pltpu.VMEM((_SUP, 128), jnp.int32),
                   pltpu.VMEM((_SUP, 128), jnp.int32),
                   pltpu.VMEM((128, D), F32),
                   pltpu.VMEM((ZB, D), F32),
                   pltpu.VMEM_SHARED((T + 8, D), F32),
                   pltpu.SemaphoreType.DMA]
    else:
        scratch = [pltpu.VMEM((_SUP, 128), jnp.int32),
                   pltpu.VMEM((128, D), F32),
                   pltpu.VMEM((ZB, D), F32),
                   pltpu.VMEM_SHARED((T + 8, D), F32),
                   pltpu.SemaphoreType.DMA]
    f = pl.kernel(
        body,
        mesh=mesh,
        out_type=jax.ShapeDtypeStruct((2 * T, D), F32),
        scratch_types=scratch,
    )
    zeros_blk = jnp.zeros((ZB, D), F32)
    if gather:
        return f(table2, g0, g1, sidx, zeros_blk)
    ones_blk = jnp.ones((128, D), F32)
    return f(ones_blk, sidx, zeros_blk)


def _pad_idx(idx, mp, fill):
    m = idx.shape[0]
    return jnp.concatenate(
        [idx, jnp.full((mp - m,), fill, jnp.int32)]).reshape(-1, 128)


def _dot(a, b):
    return jnp.dot(a, b, preferred_element_type=F32)


def _tc_input_proj(x, w, b):
    N, Din = x.shape
    H = w.shape[1]
    R = 1000

    def body(x_ref, w_ref, b_ref, o_ref):
        o_ref[...] = jax.nn.relu(_dot(x_ref[...], w_ref[...]) + b_ref[...])

    return pl.pallas_call(
        body,
        grid=(N // R,),
        in_specs=[pl.BlockSpec((R, Din), lambda i: (i, 0)),
                  pl.BlockSpec((Din, H), lambda i: (0, 0)),
                  pl.BlockSpec((1, H), lambda i: (0, 0))],
        out_specs=pl.BlockSpec((R, H), lambda i: (i, 0)),
        out_shape=jax.ShapeDtypeStruct((N, H), F32),
    )(x, w, b.reshape(1, H))


def _tc_n2e(h, p):
    N, H = h.shape
    R = 1000

    def body(h_ref, w1, b1, g, beta, w2, b2, o_ref):
        v = _dot(h_ref[...], w1[...]) + b1[...]
        v = jax.nn.relu(_ln(v, g[...], beta[...]))
        mm = _dot(v, w2[...]) + b2[...]
        o_ref[0] = mm[:, :H // 2]
        o_ref[1] = mm[:, H // 2:]

    return pl.pallas_call(
        body,
        grid=(N // R,),
        in_specs=[pl.BlockSpec((R, H), lambda i: (i, 0))] +
                 [pl.BlockSpec((H, H), lambda i: (0, 0)),
                  pl.BlockSpec((1, H), lambda i: (0, 0)),
                  pl.BlockSpec((1, H), lambda i: (0, 0)),
                  pl.BlockSpec((1, H), lambda i: (0, 0)),
                  pl.BlockSpec((H, H), lambda i: (0, 0)),
                  pl.BlockSpec((1, H), lambda i: (0, 0))],
        out_specs=pl.BlockSpec((2, R, H // 2), lambda i: (0, i, 0)),
        out_shape=jax.ShapeDtypeStruct((2, N, H // 2), F32),
    )(h, p['W1'], p['b1'].reshape(1, H), p['g'].reshape(1, H),
      p['beta'].reshape(1, H), p['W2'], p['b2'].reshape(1, H))


def _tc_e2n(he_attr, agg2, cnt, p):
    E, Dhe = he_attr.shape
    H = agg2.shape[2] * 2
    R = 1000
    w1a = p['W1'][:Dhe]
    w1b = p['W1'][Dhe:]

    def body(ha, a2, c_ref, w1a_ref, w1b_ref, b1, g, beta, w2, b2, o_ref):
        a = jnp.concatenate([a2[0], a2[1]], axis=-1)
        a = a / (c_ref[...] + 1e-6)
        v = _dot(ha[...], w1a_ref[...]) + _dot(a, w1b_ref[...]) + b1[...]
        v = jax.nn.relu(_ln(v, g[...], beta[...]))
        mm = jax.nn.relu(_dot(v, w2[...]) + b2[...])
        o_ref[0] = mm[:, :H // 2]
        o_ref[1] = mm[:, H // 2:]

    return pl.pallas_call(
        body,
        grid=(E // R,),
        in_specs=[pl.BlockSpec((R, Dhe), lambda i: (i, 0)),
                  pl.BlockSpec((2, R, H // 2), lambda i: (0, i, 0)),
                  pl.BlockSpec((R, 1), lambda i: (i, 0)),
                  pl.BlockSpec((Dhe, H), lambda i: (0, 0)),
                  pl.BlockSpec((H, H), lambda i: (0, 0)),
                  pl.BlockSpec((1, H), lambda i: (0, 0)),
                  pl.BlockSpec((1, H), lambda i: (0, 0)),
                  pl.BlockSpec((1, H), lambda i: (0, 0)),
                  pl.BlockSpec((H, H), lambda i: (0, 0)),
                  pl.BlockSpec((1, H), lambda i: (0, 0))],
        out_specs=pl.BlockSpec((2, R, H // 2), lambda i: (0, i, 0)),
        out_shape=jax.ShapeDtypeStruct((2, E, H // 2), F32),
    )(he_attr, agg2, cnt, w1a, w1b, p['b1'].reshape(1, H),
      p['g'].reshape(1, H), p['beta'].reshape(1, H), p['W2'],
      p['b2'].reshape(1, H))


def _tc_post(h, out2, deg2, g, b):
    N, H = h.shape
    R = 1000

    def body(h_ref, o2, d2, g_ref, b_ref, o_ref):
        o = jnp.concatenate([o2[0], o2[1]], axis=-1)
        deg = d2[0] + d2[1]
        o = o / (deg + 1e-6)
        o_ref[...] = h_ref[...] + _ln(o, g_ref[...], b_ref[...])

    return pl.pallas_call(
        body,
        grid=(N // R,),
        in_specs=[pl.BlockSpec((R, H), lambda i: (i, 0)),
                  pl.BlockSpec((2, R, H // 2), lambda i: (0, i, 0)),
                  pl.BlockSpec((2, R, 1), lambda i: (0, i, 0)),
                  pl.BlockSpec((1, H), lambda i: (0, 0)),
                  pl.BlockSpec((1, H), lambda i: (0, 0))],
        out_specs=pl.BlockSpec((R, H), lambda i: (i, 0)),
        out_shape=jax.ShapeDtypeStruct((N, H), F32),
    )(h, out2, deg2, g.reshape(1, H), b.reshape(1, H))


def _onehot(bat, G):
    R = bat.shape[0]
    return (bat == lax.broadcasted_iota(jnp.int32, (R, G), 1)).astype(F32)


def _sel_mask(npos, bat, resp, anym, G):
    rsel = _dot(_onehot(bat, G), resp)
    mask = (npos < rsel).astype(F32)
    return jnp.maximum(mask, 1.0 - anym)


def _tc_any_mask(npos, bat, resp, G):
    N = npos.shape[0]
    R = 1000

    def body(np_ref, b_ref, r_ref, o_ref):
        i = pl.program_id(0)
        rsel = _dot(_onehot(b_ref[...], G), r_ref[...])
        mask = (np_ref[...] < rsel).astype(F32)

        @pl.when(i == 0)
        def _():
            o_ref[...] = jnp.zeros_like(o_ref)

        o_ref[...] = jnp.maximum(o_ref[...],
                                 jnp.max(mask, axis=(0, 1), keepdims=True))

    return pl.pallas_call(
        body,
        grid=(N // R,),
        in_specs=[pl.BlockSpec((R, 1), lambda i: (i, 0)),
                  pl.BlockSpec((R, 1), lambda i: (i, 0)),
                  pl.BlockSpec((G, 1), lambda i: (0, 0))],
        out_specs=pl.BlockSpec((1, 1), lambda i: (0, 0)),
        out_shape=jax.ShapeDtypeStruct((1, 1), F32),
    )(npos, bat, resp)


def _tc_graph_key(h, npos, bat, resp, anym, wk, bk, G):
    N, H = h.shape
    R = 1000
    ngrid = N // R

    def body(h_ref, np_ref, b_ref, r_ref, a_ref, wk_ref, bk_ref, k_ref,
             hsum, cnts):
        i = pl.program_id(0)

        @pl.when(i == 0)
        def _():
            hsum[...] = jnp.zeros_like(hsum)
            cnts[...] = jnp.zeros_like(cnts)

        oneh = _onehot(b_ref[...], G)
        sel = _sel_mask(np_ref[...], b_ref[...], r_ref[...], a_ref[...], G)
        dn = (((0,), (0,)), ((), ()))
        cnts[...] += lax.dot_general(oneh, sel, dn,
                                     preferred_element_type=F32)
        hsum[...] += lax.dot_general(oneh, h_ref[...] * sel, dn,
                                     preferred_element_type=F32)

        @pl.when(i == ngrid - 1)
        def _():
            hm = hsum[...] / (cnts[...] + 1e-6)
            k_ref[...] = _dot(hm, wk_ref[...]) + bk_ref[...]

    return pl.pallas_call(
        body,
        grid=(ngrid,),
        in_specs=[pl.BlockSpec((R, H), lambda i: (i, 0)),
                  pl.BlockSpec((R, 1), lambda i: (i, 0)),
                  pl.BlockSpec((R, 1), lambda i: (i, 0)),
                  pl.BlockSpec((G, 1), lambda i: (0, 0)),
                  pl.BlockSpec((1, 1), lambda i: (0, 0)),
                  pl.BlockSpec((H, H), lambda i: (0, 0)),
                  pl.BlockSpec((1, H), lambda i: (0, 0))],
        out_specs=pl.BlockSpec((G, H), lambda i: (0, 0)),
        out_shape=jax.ShapeDtypeStruct((G, H), F32),
        scratch_shapes=[pltpu.VMEM((G, H), F32), pltpu.VMEM((G, 1), F32)],
    )(h, npos, bat, resp, anym, wk, bk.reshape(1, H))


def _tc_scores(h, npos, bat, resp, anym, k, wq, bq, G):
    N, H = h.shape
    R = 1000

    def body(h_ref, np_ref, b_ref, r_ref, a_ref, k_ref, wq_ref, bq_ref,
             sc_ref, smax_ref):
        i = pl.program_id(0)
        oneh = _onehot(b_ref[...], G)
        sel = _sel_mask(np_ref[...], b_ref[...], r_ref[...], a_ref[...], G)
        q = _dot(h_ref[...], wq_ref[...]) + bq_ref[...]
        kb = _dot(oneh, k_ref[...])
        sc = jnp.sum(q * kb, axis=-1, keepdims=True)
        sc_ref[...] = sc
        masked = jnp.where(sel > 0, sc, -jnp.inf)

        @pl.when(i == 0)
        def _():
            smax_ref[...] = jnp.full_like(smax_ref, -jnp.inf)

        smax_ref[...] = jnp.maximum(
            smax_ref[...], jnp.max(masked, axis=(0, 1), keepdims=True))

    return pl.pallas_call(
        body,
        grid=(N // R,),
        in_specs=[pl.BlockSpec((R, H), lambda i: (i, 0)),
                  pl.BlockSpec((R, 1), lambda i: (i, 0)),
                  pl.BlockSpec((R, 1), lambda i: (i, 0)),
                  pl.BlockSpec((G, 1), lambda i: (0, 0)),
                  pl.BlockSpec((1, 1), lambda i: (0, 0)),
                  pl.BlockSpec((G, H), lambda i: (0, 0)),
                  pl.BlockSpec((H, H), lambda i: (0, 0)),
                  pl.BlockSpec((1, H), lambda i: (0, 0))],
        out_specs=[pl.BlockSpec((R, 1), lambda i: (i, 0)),
                   pl.BlockSpec((1, 1), lambda i: (0, 0))],
        out_shape=[jax.ShapeDtypeStruct((N, 1), F32),
                   jax.ShapeDtypeStruct((1, 1), F32)],
    )(h, npos, bat, resp, anym, k, wq, bq.reshape(1, H))


def _tc_denom(scores, npos, bat, resp, anym, smax, G):
    N = scores.shape[0]

    def body(sc_ref, np_ref, b_ref, r_ref, a_ref, sm_ref, o_ref):
        oneh = _onehot(b_ref[...], G)
        sel = _sel_mask(np_ref[...], b_ref[...], r_ref[...], a_ref[...], G)
        exp_s = jnp.where(sel > 0, jnp.exp(sc_ref[...] - sm_ref[...]), 0.0)
        dn = (((0,), (0,)), ((), ()))
        o_ref[...] = lax.dot_general(oneh, exp_s, dn,
                                     preferred_element_type=F32) + 1e-8

    return pl.pallas_call(
        body,
        grid=(1,),
        in_specs=[pl.BlockSpec((N, 1), lambda i: (0, 0)),
                  pl.BlockSpec((N, 1), lambda i: (0, 0)),
                  pl.BlockSpec((N, 1), lambda i: (0, 0)),
                  pl.BlockSpec((G, 1), lambda i: (0, 0)),
                  pl.BlockSpec((1, 1), lambda i: (0, 0)),
                  pl.BlockSpec((1, 1), lambda i: (0, 0))],
        out_specs=pl.BlockSpec((G, 1), lambda i: (0, 0)),
        out_shape=jax.ShapeDtypeStruct((G, 1), F32),
    )(scores, npos, bat, resp, anym, smax)


def _tc_pool_logits(h, scores, npos, bat, resp, anym, smax, denom,
                    wc1, bc1, wc2, bc2, G):
    N, H = h.shape
    R = 1000
    ngrid = N // R
    Hc = wc1.shape[1]

    def body(h_ref, sc_ref, np_ref, b_ref, r_ref, a_ref, sm_ref, d_ref,
             wc1_ref, bc1_ref, wc2_ref, bc2_ref, o_ref, hg):
        i = pl.program_id(0)

        @pl.when(i == 0)
        def _():
            hg[...] = jnp.zeros_like(hg)

        oneh = _onehot(b_ref[...], G)
        sel = _sel_mask(np_ref[...], b_ref[...], r_ref[...], a_ref[...], G)
        exp_s = jnp.where(sel > 0, jnp.exp(sc_ref[...] - sm_ref[...]), 0.0)
        att = exp_s / _dot(oneh, d_ref[...])
        dn = (((0,), (0,)), ((), ()))
        hg[...] += lax.dot_general(oneh, h_ref[...] * att, dn,
                                   preferred_element_type=F32)

        @pl.when(i == ngrid - 1)
        def _():
            hc = jax.nn.relu(_dot(hg[...], wc1_ref[...]) + bc1_ref[...])
            o_ref[...] = _dot(hc, wc2_ref[...]) + bc2_ref[...]

    return pl.pallas_call(
        body,
        grid=(ngrid,),
        in_specs=[pl.BlockSpec((R, H), lambda i: (i, 0)),
                  pl.BlockSpec((R, 1), lambda i: (i, 0)),
                  pl.BlockSpec((R, 1), lambda i: (i, 0)),
                  pl.BlockSpec((R, 1), lambda i: (i, 0)),
                  pl.BlockSpec((G, 1), lambda i: (0, 0)),
                  pl.BlockSpec((1, 1), lambda i: (0, 0)),
                  pl.BlockSpec((1, 1), lambda i: (0, 0)),
                  pl.BlockSpec((G, 1), lambda i: (0, 0)),
                  pl.BlockSpec((H, Hc), lambda i: (0, 0)),
                  pl.BlockSpec((1, Hc), lambda i: (0, 0)),
                  pl.BlockSpec((Hc, 1), lambda i: (0, 0)),
                  pl.BlockSpec((1, 1), lambda i: (0, 0))],
        out_specs=pl.BlockSpec((G, 1), lambda i: (0, 0)),
        out_shape=jax.ShapeDtypeStruct((G, 1), F32),
        scratch_shapes=[pltpu.VMEM((G, H), F32)],
    )(h, scores, npos, bat, resp, anym, smax, denom,
      wc1, bc1.reshape(1, Hc), wc2, bc2.reshape(1, 1))


def kernel(x, he_index, he_attr, he_count, node_pos, response_idx, batch,
           params):
    N = x.shape[0]
    E, Dhe = he_attr.shape
    G = response_idx.shape[0]
    H = params['W_in'].shape[1]
    M = he_index.shape[1]

    node_ids = he_index[0]
    he_ids = he_index[1]

    mr = -(-(-(-M // 128)) // 256) * 256
    mp = mr * 128
    g_node0 = _pad_idx(node_ids, mp, 0)
    g_node1 = g_node0 + N
    g_he0 = _pad_idx(he_ids, mp, 0)
    g_he1 = g_he0 + E
    s_node = _pad_idx(node_ids, mp, N)
    s_he = _pad_idx(he_ids, mp, E)

    deg2 = _sc_scatter_add(None, None, None, s_node, N, H // 2,
                           split_by_core=True
                           ).reshape(2, N, H // 2)[:, :, 0:1]

    h = _tc_input_proj(x, params['W_in'], params['b_in'])
    cnt = he_count.reshape(E, 1)

    for lp in params['layers']:
        m2 = _tc_n2e(h, lp['n2e'])
        agg2 = _sc_scatter_add(m2.reshape(2 * N, H // 2), g_node0, g_node1,
                               s_he, E, H // 2,
                               split_by_core=False).reshape(2, E, H // 2)
        inc2 = _tc_e2n(he_attr, agg2, cnt, lp['e2n'])
        out2 = _sc_scatter_add(inc2.reshape(2 * E, H // 2), g_he0, g_he1,
                               s_node, N, H // 2,
                               split_by_core=False).reshape(2, N, H // 2)
        h = _tc_post(h, out2, deg2, lp['ln_g'], lp['ln_b'])

    npos = node_pos.astype(F32).reshape(N, 1)
    bat = batch.reshape(N, 1)
    resp = response_idx.astype(F32).reshape(G, 1)

    anym = _tc_any_mask(npos, bat, resp, G)
    k = _tc_graph_key(h, npos, bat, resp, anym, params['Wk'], params['bk'], G)
    scores, smax = _tc_scores(h, npos, bat, resp, anym, k,
                              params['Wq'], params['bq'], G)
    denom = _tc_denom(scores, npos, bat, resp, anym, smax, G)
    logits = _tc_pool_logits(h, scores, npos, bat, resp, anym, smax, denom,
                             params['Wc1'], params['bc1'],
                             params['Wc2'], params['bc2'], G)
    return logits.reshape(-1)


# speedup vs baseline: 3.3091x; 1.0211x over previous
import jax
import jax.numpy as jnp
from jax import lax
from jax.experimental import pallas as pl
from jax.experimental.pallas import tpu as pltpu
from jax.experimental.pallas import tpu_sc as plsc

F32 = jnp.float32


def _ln(v, g, b):
    mu = jnp.mean(v, axis=-1, keepdims=True)
    var = jnp.mean((v - mu) ** 2, axis=-1, keepdims=True)
    return (v - mu) / jnp.sqrt(var + 1e-5) * g + b


_NS = 16


def _sc_scatter_add(table2, g0, g1, sidx, T, D, split_by_core):
    MR = sidx.shape[0]
    rows_per_tile = MR // (2 * _NS) if split_by_core else MR // _NS
    _SUP = 8 if split_by_core else 16
    n_sup = rows_per_tile // _SUP
    ZB = 40
    NB = T // ZB
    nzi = -(-NB // _NS)
    gather = table2 is not None

    def body(*refs):
        if gather:
            (table_ref, g0_ref, g1_ref, sidx_ref, zeros_ref, out_ref,
             gvx, svx, rows0, rows1, zbuf, acc, semg0, semg1, sems) = refs
            rows_bufs = (rows0, rows1)
            gsems = (semg0, semg1)
        else:
            (ones_ref, sidx_ref, zeros_ref, out_ref,
             svx, rows, zbuf, acc, semg, sems) = refs
        c = lax.axis_index("c")
        s = lax.axis_index("s")

        if not gather:
            pltpu.sync_copy(ones_ref, rows)

        pltpu.sync_copy(zeros_ref, zbuf)

        def zblk(i, carry):
            b = s + _NS * i

            @pl.when(b < NB)
            def _():
                pltpu.sync_copy(zbuf, acc.at[pl.ds(b * ZB, ZB)])

            return carry

        lax.fori_loop(0, nzi, zblk, 0)
        plsc.subcore_barrier()

        if split_by_core:
            tile_row0 = (c * _NS + s) * rows_per_tile
        else:
            tile_row0 = s * rows_per_tile

        def sup_body(j, carry):
            r0 = tile_row0 + j * _SUP

            if gather:
                @pl.when(c == 0)
                def _():
                    pltpu.sync_copy(g0_ref.at[pl.ds(r0, _SUP)], gvx)

                @pl.when(c == 1)
                def _():
                    pltpu.sync_copy(g1_ref.at[pl.ds(r0, _SUP)], gvx)

            pltpu.sync_copy(sidx_ref.at[pl.ds(r0, _SUP)], svx)
            if gather:
                ga = [None] * _SUP
                sc = [None] * _SUP
                ga[0] = pltpu.async_copy(table_ref.at[gvx.at[0]],
                                         rows_bufs[0], gsems[0])
                for b in range(_SUP):
                    if b + 1 < _SUP:
                        if b >= 1:
                            sc[b - 1].wait()
                        ga[b + 1] = pltpu.async_copy(
                            table_ref.at[gvx.at[b + 1]],
                            rows_bufs[(b + 1) % 2], gsems[(b + 1) % 2])
                    ga[b].wait()
                    sc[b] = pltpu.async_copy(rows_bufs[b % 2],
                                             acc.at[svx.at[b]], sems,
                                             add=True)
                sc[_SUP - 2].wait()
                sc[_SUP - 1].wait()
            else:
                for b in range(_SUP):
                    pltpu.sync_copy(rows, acc.at[svx.at[b]], add=True)
            return carry

        lax.fori_loop(0, n_sup, sup_body, 0)
        plsc.subcore_barrier()

        def wblk(i, carry):
            b = s + _NS * i

            @pl.when(b < NB)
            def _():
                pltpu.sync_copy(acc.at[pl.ds(b * ZB, ZB)],
                                out_ref.at[pl.ds(c * T + b * ZB, ZB)])

            return carry

        lax.fori_loop(0, nzi, wblk, 0)

    mesh = plsc.VectorSubcoreMesh(core_axis_name="c", subcore_axis_name="s")
    if gather:
        scratch = [pltpu.VMEM((_SUP, 128), jnp.int32),
                   pltpu.VMEM((_SUP, 128), jnp.int32),
                   pltpu.VMEM((128, D), F32),
                   pltpu.VMEM((128, D), F32),
                   pltpu.VMEM((ZB, D), F32),
                   pltpu.VMEM_SHARED((T + 8, D), F32),
                   pltpu.SemaphoreType.DMA,
                   pltpu.SemaphoreType.DMA,
                   pltpu.SemaphoreType.DMA]
    else:
        scratch = [pltpu.VMEM((_SUP, 128), jnp.int32),
                   pltpu.VMEM((128, D), F32),
                   pltpu.VMEM((ZB, D), F32),
                   pltpu.VMEM_SHARED((T + 8, D), F32),
                   pltpu.SemaphoreType.DMA,
                   pltpu.SemaphoreType.DMA]
    f = pl.kernel(
        body,
        mesh=mesh,
        out_type=jax.ShapeDtypeStruct((2 * T, D), F32),
        scratch_types=scratch,
    )
    zeros_blk = jnp.zeros((ZB, D), F32)
    if gather:
        return f(table2, g0, g1, sidx, zeros_blk)
    ones_blk = jnp.ones((128, D), F32)
    return f(ones_blk, sidx, zeros_blk)


def _pad_idx(idx, mp, fill):
    m = idx.shape[0]
    return jnp.concatenate(
        [idx, jnp.full((mp - m,), fill, jnp.int32)]).reshape(-1, 128)


def _dot(a, b):
    return jnp.dot(a, b, preferred_element_type=F32)


def _dotg(a, b):
    return jnp.dot(a, b, preferred_element_type=F32,
                   precision=lax.Precision.HIGHEST)


def _tc_input_proj(x, w, b):
    N, Din = x.shape
    H = w.shape[1]
    R = 1000

    def body(x_ref, w_ref, b_ref, o_ref):
        o_ref[...] = jax.nn.relu(_dot(x_ref[...], w_ref[...]) + b_ref[...])

    return pl.pallas_call(
        body,
        grid=(N // R,),
        in_specs=[pl.BlockSpec((R, Din), lambda i: (i, 0)),
                  pl.BlockSpec((Din, H), lambda i: (0, 0)),
                  pl.BlockSpec((1, H), lambda i: (0, 0))],
        out_specs=pl.BlockSpec((R, H), lambda i: (i, 0)),
        out_shape=jax.ShapeDtypeStruct((N, H), F32),
    )(x, w, b.reshape(1, H))


def _ln_stats(v):
    mu = jnp.mean(v, axis=-1, keepdims=True)
    var = jnp.mean((v - mu) ** 2, axis=-1, keepdims=True)
    return mu, var


def _ln_core(v, mu, var, g, b):
    return (v - mu) / jnp.sqrt(var + 1e-5) * g + b


def _tc_dot_bias(x, w, b):
    n, K = x.shape
    H = w.shape[1]
    R = 1000

    def body(x_ref, w_ref, b_ref, o_ref):
        o_ref[...] = _dot(x_ref[...], w_ref[...]) + b_ref[...]

    return pl.pallas_call(
        body,
        grid=(n // R,),
        in_specs=[pl.BlockSpec((R, K), lambda i: (i, 0)),
                  pl.BlockSpec((K, H), lambda i: (0, 0)),
                  pl.BlockSpec((1, H), lambda i: (0, 0))],
        out_specs=pl.BlockSpec((R, H), lambda i: (i, 0)),
        out_shape=jax.ShapeDtypeStruct((n, H), F32),
    )(x, w, b.reshape(1, H))


def _tc_mlp_out(v, mu, var, g, beta, w2, b2, final_relu):
    n, H = v.shape
    R = 1000

    def body(v_ref, mu_ref, var_ref, g_ref, be_ref, w2_ref, b2_ref, o_ref):
        vn = jax.nn.relu(_ln_core(v_ref[...], mu_ref[...], var_ref[...],
                                  g_ref[...], be_ref[...]))
        mm = _dot(vn, w2_ref[...]) + b2_ref[...]
        if final_relu:
            mm = jax.nn.relu(mm)
        o_ref[0] = mm[:, :H // 2]
        o_ref[1] = mm[:, H // 2:]

    return pl.pallas_call(
        body,
        grid=(n // R,),
        in_specs=[pl.BlockSpec((R, H), lambda i: (i, 0)),
                  pl.BlockSpec((R, 1), lambda i: (i, 0)),
                  pl.BlockSpec((R, 1), lambda i: (i, 0)),
                  pl.BlockSpec((1, H), lambda i: (0, 0)),
                  pl.BlockSpec((1, H), lambda i: (0, 0)),
                  pl.BlockSpec((H, H), lambda i: (0, 0)),
                  pl.BlockSpec((1, H), lambda i: (0, 0))],
        out_specs=pl.BlockSpec((2, R, H // 2), lambda i: (0, i, 0)),
        out_shape=jax.ShapeDtypeStruct((2, n, H // 2), F32),
    )(v, mu, var, g.reshape(1, H), beta.reshape(1, H), w2, b2.reshape(1, H))


def _tc_n2e(h, p):
    v = _tc_dot_bias(h, p['W1'], p['b1'])
    mu, var = _ln_stats(v)
    return _tc_mlp_out(v, mu, var, p['g'], p['beta'], p['W2'], p['b2'],
                       final_relu=False)


def _tc_e2n_pre(he_attr, agg2, cnt, w1a, w1b, b1):
    E, Dhe = he_attr.shape
    H = agg2.shape[2] * 2
    R = 1000

    def body(ha, a2, c_ref, w1a_ref, w1b_ref, b1_ref, o_ref):
        a = jnp.concatenate([a2[0], a2[1]], axis=-1)
        a = a / (c_ref[...] + 1e-6)
        o_ref[...] = (_dot(ha[...], w1a_ref[...]) + _dot(a, w1b_ref[...])
                      + b1_ref[...])

    return pl.pallas_call(
        body,
        grid=(E // R,),
        in_specs=[pl.BlockSpec((R, Dhe), lambda i: (i, 0)),
                  pl.BlockSpec((2, R, H // 2), lambda i: (0, i, 0)),
                  pl.BlockSpec((R, 1), lambda i: (i, 0)),
                  pl.BlockSpec((Dhe, H), lambda i: (0, 0)),
                  pl.BlockSpec((H, H), lambda i: (0, 0)),
                  pl.BlockSpec((1, H), lambda i: (0, 0))],
        out_specs=pl.BlockSpec((R, H), lambda i: (i, 0)),
        out_shape=jax.ShapeDtypeStruct((E, H), F32),
    )(he_attr, agg2, cnt, w1a, w1b, b1.reshape(1, H))


def _tc_e2n(he_attr, agg2, cnt, p):
    Dhe = he_attr.shape[1]
    v = _tc_e2n_pre(he_attr, agg2, cnt, p['W1'][:Dhe], p['W1'][Dhe:],
                    p['b1'])
    mu, var = _ln_stats(v)
    return _tc_mlp_out(v, mu, var, p['g'], p['beta'], p['W2'], p['b2'],
                       final_relu=True)


def _tc_post_pre(out2, deg2):
    n = out2.shape[1]
    H = out2.shape[2] * 2
    R = 1000

    def body(o2, d2, o_ref):
        o = jnp.concatenate([o2[0], o2[1]], axis=-1)
        deg = d2[0] + d2[1]
        o_ref[...] = o / (deg + 1e-6)

    return pl.pallas_call(
        body,
        grid=(n // R,),
        in_specs=[pl.BlockSpec((2, R, H // 2), lambda i: (0, i, 0)),
                  pl.BlockSpec((2, R, 1), lambda i: (0, i, 0))],
        out_specs=pl.BlockSpec((R, H), lambda i: (i, 0)),
        out_shape=jax.ShapeDtypeStruct((n, H), F32),
    )(out2, deg2)


def _tc_post(h, out2, deg2, g, b):
    N, H = h.shape
    R = 1000
    o = _tc_post_pre(out2, deg2)
    mu, var = _ln_stats(o)

    def body(h_ref, o_ref_in, mu_ref, var_ref, g_ref, b_ref, o_ref):
        o_ref[...] = h_ref[...] + _ln_core(o_ref_in[...], mu_ref[...],
                                           var_ref[...], g_ref[...],
                                           b_ref[...])

    return pl.pallas_call(
        body,
        grid=(N // R,),
        in_specs=[pl.BlockSpec((R, H), lambda i: (i, 0)),
                  pl.BlockSpec((R, H), lambda i: (i, 0)),
                  pl.BlockSpec((R, 1), lambda i: (i, 0)),
                  pl.BlockSpec((R, 1), lambda i: (i, 0)),
                  pl.BlockSpec((1, H), lambda i: (0, 0)),
                  pl.BlockSpec((1, H), lambda i: (0, 0))],
        out_specs=pl.BlockSpec((R, H), lambda i: (i, 0)),
        out_shape=jax.ShapeDtypeStruct((N, H), F32),
    )(h, o, mu, var, g.reshape(1, H), b.reshape(1, H))


def _onehot(bat, G):
    R = bat.shape[0]
    return (bat == lax.broadcasted_iota(jnp.int32, (R, G), 1)).astype(F32)


def _sel_mask(npos, bat, resp, anym, G):
    rsel = _dotg(_onehot(bat, G), resp)
    mask = (npos < rsel).astype(F32)
    return jnp.maximum(mask, 1.0 - anym)


def _tc_any_mask(npos, bat, resp, G):
    N = npos.shape[0]
    R = 1000

    def body(np_ref, b_ref, r_ref, o_ref):
        i = pl.program_id(0)
        rsel = _dotg(_onehot(b_ref[...], G), r_ref[...])
        mask = (np_ref[...] < rsel).astype(F32)

        @pl.when(i == 0)
        def _():
            o_ref[...] = jnp.zeros_like(o_ref)

        o_ref[...] = jnp.maximum(o_ref[...],
                                 jnp.max(mask, axis=(0, 1), keepdims=True))

    return pl.pallas_call(
        body,
        grid=(N // R,),
        in_specs=[pl.BlockSpec((R, 1), lambda i: (i, 0)),
                  pl.BlockSpec((R, 1), lambda i: (i, 0)),
                  pl.BlockSpec((G, 1), lambda i: (0, 0))],
        out_specs=pl.BlockSpec((1, 1), lambda i: (0, 0)),
        out_shape=jax.ShapeDtypeStruct((1, 1), F32),
    )(npos, bat, resp)


def _tc_graph_key(h, npos, bat, resp, anym, wk, bk, G):
    N, H = h.shape
    R = 1000
    ngrid = N // R

    def body(h_ref, np_ref, b_ref, r_ref, a_ref, wk_ref, bk_ref, k_ref,
             hsum, cnts):
        i = pl.program_id(0)

        @pl.when(i == 0)
        def _():
            hsum[...] = jnp.zeros_like(hsum)
            cnts[...] = jnp.zeros_like(cnts)

        oneh = _onehot(b_ref[...], G)
        sel = _sel_mask(np_ref[...], b_ref[...], r_ref[...], a_ref[...], G)
        dn = (((0,), (0,)), ((), ()))
        cnts[...] += lax.dot_general(oneh, sel, dn,
                                     preferred_element_type=F32,
                                     precision=lax.Precision.HIGHEST)
        hsum[...] += lax.dot_general(oneh, h_ref[...] * sel, dn,
                                     preferred_element_type=F32,
                                     precision=lax.Precision.HIGHEST)

        @pl.when(i == ngrid - 1)
        def _():
            hm = hsum[...] / (cnts[...] + 1e-6)
            k_ref[...] = _dot(hm, wk_ref[...]) + bk_ref[...]

    return pl.pallas_call(
        body,
        grid=(ngrid,),
        in_specs=[pl.BlockSpec((R, H), lambda i: (i, 0)),
                  pl.BlockSpec((R, 1), lambda i: (i, 0)),
                  pl.BlockSpec((R, 1), lambda i: (i, 0)),
                  pl.BlockSpec((G, 1), lambda i: (0, 0)),
                  pl.BlockSpec((1, 1), lambda i: (0, 0)),
                  pl.BlockSpec((H, H), lambda i: (0, 0)),
                  pl.BlockSpec((1, H), lambda i: (0, 0))],
        out_specs=pl.BlockSpec((G, H), lambda i: (0, 0)),
        out_shape=jax.ShapeDtypeStruct((G, H), F32),
        scratch_shapes=[pltpu.VMEM((G, H), F32), pltpu.VMEM((G, 1), F32)],
    )(h, npos, bat, resp, anym, wk, bk.reshape(1, H))


def _tc_scores(h, npos, bat, resp, anym, k, wq, bq, G):
    N, H = h.shape
    R = 1000

    def body(h_ref, np_ref, b_ref, r_ref, a_ref, k_ref, wq_ref, bq_ref,
             sc_ref, smax_ref):
        i = pl.program_id(0)
        oneh = _onehot(b_ref[...], G)
        sel = _sel_mask(np_ref[...], b_ref[...], r_ref[...], a_ref[...], G)
        q = _dot(h_ref[...], wq_ref[...]) + bq_ref[...]
        kb = _dotg(oneh, k_ref[...])
        sc = jnp.sum(q * kb, axis=-1, keepdims=True)
        sc_ref[...] = sc
        masked = jnp.where(sel > 0, sc, -jnp.inf)

        @pl.when(i == 0)
        def _():
            smax_ref[...] = jnp.full_like(smax_ref, -jnp.inf)

        smax_ref[...] = jnp.maximum(
            smax_ref[...], jnp.max(masked, axis=(0, 1), keepdims=True))

    return pl.pallas_call(
        body,
        grid=(N // R,),
        in_specs=[pl.BlockSpec((R, H), lambda i: (i, 0)),
                  pl.BlockSpec((R, 1), lambda i: (i, 0)),
                  pl.BlockSpec((R, 1), lambda i: (i, 0)),
                  pl.BlockSpec((G, 1), lambda i: (0, 0)),
                  pl.BlockSpec((1, 1), lambda i: (0, 0)),
                  pl.BlockSpec((G, H), lambda i: (0, 0)),
                  pl.BlockSpec((H, H), lambda i: (0, 0)),
                  pl.BlockSpec((1, H), lambda i: (0, 0))],
        out_specs=[pl.BlockSpec((R, 1), lambda i: (i, 0)),
                   pl.BlockSpec((1, 1), lambda i: (0, 0))],
        out_shape=[jax.ShapeDtypeStruct((N, 1), F32),
                   jax.ShapeDtypeStruct((1, 1), F32)],
    )(h, npos, bat, resp, anym, k, wq, bq.reshape(1, H))


def _tc_denom(scores, npos, bat, resp, anym, smax, G):
    N = scores.shape[0]

    def body(sc_ref, np_ref, b_ref, r_ref, a_ref, sm_ref, o_ref):
        oneh = _onehot(b_ref[...], G)
        sel = _sel_mask(np_ref[...], b_ref[...], r_ref[...], a_ref[...], G)
        exp_s = jnp.where(sel > 0, jnp.exp(sc_ref[...] - sm_ref[...]), 0.0)
        dn = (((0,), (0,)), ((), ()))
        o_ref[...] = lax.dot_general(oneh, exp_s, dn,
                                     preferred_element_type=F32,
                                     precision=lax.Precision.HIGHEST) + 1e-8

    return pl.pallas_call(
        body,
        grid=(1,),
        in_specs=[pl.BlockSpec((N, 1), lambda i: (0, 0)),
                  pl.BlockSpec((N, 1), lambda i: (0, 0)),
                  pl.BlockSpec((N, 1), lambda i: (0, 0)),
                  pl.BlockSpec((G, 1), lambda i: (0, 0)),
                  pl.BlockSpec((1, 1), lambda i: (0, 0)),
                  pl.BlockSpec((1, 1), lambda i: (0, 0))],
        out_specs=pl.BlockSpec((G, 1), lambda i: (0, 0)),
        out_shape=jax.ShapeDtypeStruct((G, 1), F32),
    )(scores, npos, bat, resp, anym, smax)


def _tc_pool_logits(h, scores, npos, bat, resp, anym, smax, denom,
                    wc1, bc1, wc2, bc2, G):
    N, H = h.shape
    R = 1000
    ngrid = N // R
    Hc = wc1.shape[1]

    def body(h_ref, sc_ref, np_ref, b_ref, r_ref, a_ref, sm_ref, d_ref,
             wc1_ref, bc1_ref, wc2_ref, bc2_ref, o_ref, hg):
        i = pl.program_id(0)

        @pl.when(i == 0)
        def _():
            hg[...] = jnp.zeros_like(hg)

        oneh = _onehot(b_ref[...], G)
        sel = _sel_mask(np_ref[...], b_ref[...], r_ref[...], a_ref[...], G)
        exp_s = jnp.where(sel > 0, jnp.exp(sc_ref[...] - sm_ref[...]), 0.0)
        att = exp_s / _dotg(oneh, d_ref[...])
        dn = (((0,), (0,)), ((), ()))
        hg[...] += lax.dot_general(oneh, h_ref[...] * att, dn,
                                   preferred_element_type=F32,
                                   precision=lax.Precision.HIGHEST)

        @pl.when(i == ngrid - 1)
        def _():
            hc = jax.nn.relu(_dot(hg[...], wc1_ref[...]) + bc1_ref[...])
            o_ref[...] = _dot(hc, wc2_ref[...]) + bc2_ref[...]

    return pl.pallas_call(
        body,
        grid=(ngrid,),
        in_specs=[pl.BlockSpec((R, H), lambda i: (i, 0)),
                  pl.BlockSpec((R, 1), lambda i: (i, 0)),
                  pl.BlockSpec((R, 1), lambda i: (i, 0)),
                  pl.BlockSpec((R, 1), lambda i: (i, 0)),
                  pl.BlockSpec((G, 1), lambda i: (0, 0)),
                  pl.BlockSpec((1, 1), lambda i: (0, 0)),
                  pl.BlockSpec((1, 1), lambda i: (0, 0)),
                  pl.BlockSpec((G, 1), lambda i: (0, 0)),
                  pl.BlockSpec((H, Hc), lambda i: (0, 0)),
                  pl.BlockSpec((1, Hc), lambda i: (0, 0)),
                  pl.BlockSpec((Hc, 1), lambda i: (0, 0)),
                  pl.BlockSpec((1, 1), lambda i: (0, 0))],
        out_specs=pl.BlockSpec((G, 1), lambda i: (0, 0)),
        out_shape=jax.ShapeDtypeStruct((G, 1), F32),
        scratch_shapes=[pltpu.VMEM((G, H), F32)],
    )(h, scores, npos, bat, resp, anym, smax, denom,
      wc1, bc1.reshape(1, Hc), wc2, bc2.reshape(1, 1))


def kernel(x, he_index, he_attr, he_count, node_pos, response_idx, batch,
           params):
    N = x.shape[0]
    E, Dhe = he_attr.shape
    G = response_idx.shape[0]
    H = params['W_in'].shape[1]
    M = he_index.shape[1]

    node_ids = he_index[0]
    he_ids = he_index[1]

    mr = -(-(-(-M // 128)) // 256) * 256
    mp = mr * 128
    g_node0 = _pad_idx(node_ids, mp, 0)
    g_node1 = g_node0 + N
    g_he0 = _pad_idx(he_ids, mp, 0)
    g_he1 = g_he0 + E
    s_node = _pad_idx(node_ids, mp, N)
    s_he = _pad_idx(he_ids, mp, E)

    deg2 = _sc_scatter_add(None, None, None, s_node, N, H // 2,
                           split_by_core=True
                           ).reshape(2, N, H // 2)[:, :, 0:1]

    tok = (deg2[0, 0, 0] * 0.0).astype(jnp.int32)
    g_node0 = g_node0 + tok
    g_node1 = g_node1 + tok

    h = _tc_input_proj(x, params['W_in'], params['b_in'])
    cnt = he_count.reshape(E, 1)

    for lp in params['layers']:
        m2 = _tc_n2e(h, lp['n2e'])
        agg2 = _sc_scatter_add(m2.reshape(2 * N, H // 2), g_node0, g_node1,
                               s_he, E, H // 2,
                               split_by_core=False).reshape(2, E, H // 2)
        inc2 = _tc_e2n(he_attr, agg2, cnt, lp['e2n'])
        out2 = _sc_scatter_add(inc2.reshape(2 * E, H // 2), g_he0, g_he1,
                               s_node, N, H // 2,
                               split_by_core=False).reshape(2, N, H // 2)
        h = _tc_post(h, out2, deg2, lp['ln_g'], lp['ln_b'])

    npos = node_pos.astype(F32).reshape(N, 1)
    bat = batch.reshape(N, 1)
    resp = response_idx.astype(F32).reshape(G, 1)

    anym = _tc_any_mask(npos, bat, resp, G)
    k = _tc_graph_key(h, npos, bat, resp, anym, params['Wk'], params['bk'], G)
    scores, smax = _tc_scores(h, npos, bat, resp, anym, k,
                              params['Wq'], params['bq'], G)
    denom = _tc_denom(scores, npos, bat, resp, anym, smax, G)
    logits = _tc_pool_logits(h, scores, npos, bat, resp, anym, smax, denom,
                             params['Wc1'], params['bc1'],
                             params['Wc2'], params['bc2'], G)
    return logits.reshape(-1)


# speedup vs baseline: 3.3100x; 1.0003x over previous
import jax
import jax.numpy as jnp
from jax import lax
from jax.experimental import pallas as pl
from jax.experimental.pallas import tpu as pltpu
from jax.experimental.pallas import tpu_sc as plsc

F32 = jnp.float32


def _ln(v, g, b):
    mu = jnp.mean(v, axis=-1, keepdims=True)
    var = jnp.mean((v - mu) ** 2, axis=-1, keepdims=True)
    return (v - mu) / jnp.sqrt(var + 1e-5) * g + b


_NS = 16


def _sc_scatter_add(table2, g0, g1, sidx, T, D, split_by_core):
    MR = sidx.shape[0]
    rows_per_tile = MR // (2 * _NS) if split_by_core else MR // _NS
    _SUP = 8 if split_by_core else 16
    n_sup = rows_per_tile // _SUP
    ZB = 40
    NB = T // ZB
    nzi = -(-NB // _NS)
    gather = table2 is not None
    NBUF = 2

    def body(*refs):
        if gather:
            (table_ref, g0_ref, g1_ref, sidx_ref, zeros_ref, out_ref,
             gvx, svx, *rest) = refs
            rows_bufs = tuple(rest[:NBUF])
            gsems = tuple(rest[NBUF + 2:2 * NBUF + 2])
            ssems = tuple(rest[2 * NBUF + 2:3 * NBUF + 2])
            zbuf, acc = rest[NBUF], rest[NBUF + 1]
        else:
            (ones_ref, sidx_ref, zeros_ref, out_ref,
             svx, rows, zbuf, acc, semg, sems) = refs
        c = lax.axis_index("c")
        s = lax.axis_index("s")

        if not gather:
            pltpu.sync_copy(ones_ref, rows)

        pltpu.sync_copy(zeros_ref, zbuf)

        def zblk(i, carry):
            b = s + _NS * i

            @pl.when(b < NB)
            def _():
                pltpu.sync_copy(zbuf, acc.at[pl.ds(b * ZB, ZB)])

            return carry

        lax.fori_loop(0, nzi, zblk, 0)
        plsc.subcore_barrier()

        if split_by_core:
            tile_row0 = (c * _NS + s) * rows_per_tile
        else:
            tile_row0 = s * rows_per_tile

        def sup_body(j, carry):
            r0 = tile_row0 + j * _SUP

            if gather:
                @pl.when(c == 0)
                def _():
                    pltpu.sync_copy(g0_ref.at[pl.ds(r0, _SUP)], gvx)

                @pl.when(c == 1)
                def _():
                    pltpu.sync_copy(g1_ref.at[pl.ds(r0, _SUP)], gvx)

            pltpu.sync_copy(sidx_ref.at[pl.ds(r0, _SUP)], svx)
            if gather:
                ga = [None] * _SUP
                sc = [None] * _SUP
                ga[0] = pltpu.async_copy(table_ref.at[gvx.at[0]],
                                         rows_bufs[0], gsems[0])
                for b in range(_SUP):
                    if b + 1 < _SUP:
                        if b + 1 >= NBUF:
                            sc[b + 1 - NBUF].wait()
                        ga[b + 1] = pltpu.async_copy(
                            table_ref.at[gvx.at[b + 1]],
                            rows_bufs[(b + 1) % NBUF],
                            gsems[(b + 1) % NBUF])
                    ga[b].wait()
                    sc[b] = pltpu.async_copy(rows_bufs[b % NBUF],
                                             acc.at[svx.at[b]],
                                             ssems[b % NBUF], add=True)
                for t in range(max(0, _SUP - NBUF), _SUP):
                    sc[t].wait()
            else:
                for b in range(_SUP):
                    pltpu.sync_copy(rows, acc.at[svx.at[b]], add=True)
            return carry

        lax.fori_loop(0, n_sup, sup_body, 0)
        plsc.subcore_barrier()

        def wblk(i, carry):
            b = s + _NS * i

            @pl.when(b < NB)
            def _():
                pltpu.sync_copy(acc.at[pl.ds(b * ZB, ZB)],
                                out_ref.at[pl.ds(c * T + b * ZB, ZB)])

            return carry

        lax.fori_loop(0, nzi, wblk, 0)

    mesh = plsc.VectorSubcoreMesh(core_axis_name="c", subcore_axis_name="s")
    if gather:
        scratch = ([pltpu.VMEM((_SUP, 128), jnp.int32),
                    pltpu.VMEM((_SUP, 128), jnp.int32)]
                   + [pltpu.VMEM((128, D), F32)] * NBUF
                   + [pltpu.VMEM((ZB, D), F32),
                      pltpu.VMEM_SHARED((T + 8, D), F32)]
                   + [pltpu.SemaphoreType.DMA] * (2 * NBUF))
    else:
        scratch = [pltpu.VMEM((_SUP, 128), jnp.int32),
                   pltpu.VMEM((128, D), F32),
                   pltpu.VMEM((ZB, D), F32),
                   pltpu.VMEM_SHARED((T + 8, D), F32),
                   pltpu.SemaphoreType.DMA,
                   pltpu.SemaphoreType.DMA]
    f = pl.kernel(
        body,
        mesh=mesh,
        out_type=jax.ShapeDtypeStruct((2 * T, D), F32),
        scratch_types=scratch,
    )
    zeros_blk = jnp.zeros((ZB, D), F32)
    if gather:
        return f(table2, g0, g1, sidx, zeros_blk)
    ones_blk = jnp.ones((128, D), F32)
    return f(ones_blk, sidx, zeros_blk)


def _pad_idx(idx, mp, fill):
    m = idx.shape[0]
    return jnp.concatenate(
        [idx, jnp.full((mp - m,), fill, jnp.int32)]).reshape(-1, 128)


def _dot(a, b):
    return jnp.dot(a, b, preferred_element_type=F32)


def _dotg(a, b):
    return jnp.dot(a, b, preferred_element_type=F32,
                   precision=lax.Precision.HIGHEST)


def _tc_input_proj(x, w, b):
    N, Din = x.shape
    H = w.shape[1]
    R = 1000

    def body(x_ref, w_ref, b_ref, o_ref):
        o_ref[...] = jax.nn.relu(_dot(x_ref[...], w_ref[...]) + b_ref[...])

    return pl.pallas_call(
        body,
        grid=(N // R,),
        in_specs=[pl.BlockSpec((R, Din), lambda i: (i, 0)),
                  pl.BlockSpec((Din, H), lambda i: (0, 0)),
                  pl.BlockSpec((1, H), lambda i: (0, 0))],
        out_specs=pl.BlockSpec((R, H), lambda i: (i, 0)),
        out_shape=jax.ShapeDtypeStruct((N, H), F32),
    )(x, w, b.reshape(1, H))


def _ln_stats(v):
    mu = jnp.mean(v, axis=-1, keepdims=True)
    var = jnp.mean((v - mu) ** 2, axis=-1, keepdims=True)
    return mu, var


def _ln_core(v, mu, var, g, b):
    return (v - mu) / jnp.sqrt(var + 1e-5) * g + b


def _tc_dot_bias(x, w, b):
    n, K = x.shape
    H = w.shape[1]
    R = 1000

    def body(x_ref, w_ref, b_ref, o_ref):
        o_ref[...] = _dot(x_ref[...], w_ref[...]) + b_ref[...]

    return pl.pallas_call(
        body,
        grid=(n // R,),
        in_specs=[pl.BlockSpec((R, K), lambda i: (i, 0)),
                  pl.BlockSpec((K, H), lambda i: (0, 0)),
                  pl.BlockSpec((1, H), lambda i: (0, 0))],
        out_specs=pl.BlockSpec((R, H), lambda i: (i, 0)),
        out_shape=jax.ShapeDtypeStruct((n, H), F32),
    )(x, w, b.reshape(1, H))


def _tc_mlp_out(v, mu, var, g, beta, w2, b2, final_relu):
    n, H = v.shape
    R = 1000

    def body(v_ref, mu_ref, var_ref, g_ref, be_ref, w2_ref, b2_ref, o_ref):
        vn = jax.nn.relu(_ln_core(v_ref[...], mu_ref[...], var_ref[...],
                                  g_ref[...], be_ref[...]))
        mm = _dot(vn, w2_ref[...]) + b2_ref[...]
        if final_relu:
            mm = jax.nn.relu(mm)
        o_ref[0] = mm[:, :H // 2]
        o_ref[1] = mm[:, H // 2:]

    return pl.pallas_call(
        body,
        grid=(n // R,),
        in_specs=[pl.BlockSpec((R, H), lambda i: (i, 0)),
                  pl.BlockSpec((R, 1), lambda i: (i, 0)),
                  pl.BlockSpec((R, 1), lambda i: (i, 0)),
                  pl.BlockSpec((1, H), lambda i: (0, 0)),
                  pl.BlockSpec((1, H), lambda i: (0, 0)),
                  pl.BlockSpec((H, H), lambda i: (0, 0)),
                  pl.BlockSpec((1, H), lambda i: (0, 0))],
        out_specs=pl.BlockSpec((2, R, H // 2), lambda i: (0, i, 0)),
        out_shape=jax.ShapeDtypeStruct((2, n, H // 2), F32),
    )(v, mu, var, g.reshape(1, H), beta.reshape(1, H), w2, b2.reshape(1, H))


def _tc_n2e(h, p):
    v = _tc_dot_bias(h, p['W1'], p['b1'])
    mu, var = _ln_stats(v)
    return _tc_mlp_out(v, mu, var, p['g'], p['beta'], p['W2'], p['b2'],
                       final_relu=False)


def _tc_e2n_pre(he_attr, agg2, cnt, w1a, w1b, b1):
    E, Dhe = he_attr.shape
    H = agg2.shape[2] * 2
    R = 1000

    def body(ha, a2, c_ref, w1a_ref, w1b_ref, b1_ref, o_ref):
        a = jnp.concatenate([a2[0], a2[1]], axis=-1)
        a = a / (c_ref[...] + 1e-6)
        o_ref[...] = (_dot(ha[...], w1a_ref[...]) + _dot(a, w1b_ref[...])
                      + b1_ref[...])

    return pl.pallas_call(
        body,
        grid=(E // R,),
        in_specs=[pl.BlockSpec((R, Dhe), lambda i: (i, 0)),
                  pl.BlockSpec((2, R, H // 2), lambda i: (0, i, 0)),
                  pl.BlockSpec((R, 1), lambda i: (i, 0)),
                  pl.BlockSpec((Dhe, H), lambda i: (0, 0)),
                  pl.BlockSpec((H, H), lambda i: (0, 0)),
                  pl.BlockSpec((1, H), lambda i: (0, 0))],
        out_specs=pl.BlockSpec((R, H), lambda i: (i, 0)),
        out_shape=jax.ShapeDtypeStruct((E, H), F32),
    )(he_attr, agg2, cnt, w1a, w1b, b1.reshape(1, H))


def _tc_e2n(he_attr, agg2, cnt, p):
    Dhe = he_attr.shape[1]
    v = _tc_e2n_pre(he_attr, agg2, cnt, p['W1'][:Dhe], p['W1'][Dhe:],
                    p['b1'])
    mu, var = _ln_stats(v)
    return _tc_mlp_out(v, mu, var, p['g'], p['beta'], p['W2'], p['b2'],
                       final_relu=True)


def _tc_post_pre(out2, deg2):
    n = out2.shape[1]
    H = out2.shape[2] * 2
    R = 1000

    def body(o2, d2, o_ref):
        o = jnp.concatenate([o2[0], o2[1]], axis=-1)
        deg = d2[0] + d2[1]
        o_ref[...] = o / (deg + 1e-6)

    return pl.pallas_call(
        body,
        grid=(n // R,),
        in_specs=[pl.BlockSpec((2, R, H // 2), lambda i: (0, i, 0)),
                  pl.BlockSpec((2, R, 1), lambda i: (0, i, 0))],
        out_specs=pl.BlockSpec((R, H), lambda i: (i, 0)),
        out_shape=jax.ShapeDtypeStruct((n, H), F32),
    )(out2, deg2)


def _tc_post(h, out2, deg2, g, b):
    N, H = h.shape
    R = 1000
    o = _tc_post_pre(out2, deg2)
    mu, var = _ln_stats(o)

    def body(h_ref, o_ref_in, mu_ref, var_ref, g_ref, b_ref, o_ref):
        o_ref[...] = h_ref[...] + _ln_core(o_ref_in[...], mu_ref[...],
                                           var_ref[...], g_ref[...],
                                           b_ref[...])

    return pl.pallas_call(
        body,
        grid=(N // R,),
        in_specs=[pl.BlockSpec((R, H), lambda i: (i, 0)),
                  pl.BlockSpec((R, H), lambda i: (i, 0)),
                  pl.BlockSpec((R, 1), lambda i: (i, 0)),
                  pl.BlockSpec((R, 1), lambda i: (i, 0)),
                  pl.BlockSpec((1, H), lambda i: (0, 0)),
                  pl.BlockSpec((1, H), lambda i: (0, 0))],
        out_specs=pl.BlockSpec((R, H), lambda i: (i, 0)),
        out_shape=jax.ShapeDtypeStruct((N, H), F32),
    )(h, o, mu, var, g.reshape(1, H), b.reshape(1, H))


def _onehot(bat, G):
    R = bat.shape[0]
    return (bat == lax.broadcasted_iota(jnp.int32, (R, G), 1)).astype(F32)


def _sel_mask(npos, bat, resp, anym, G):
    rsel = _dotg(_onehot(bat, G), resp)
    mask = (npos < rsel).astype(F32)
    return jnp.maximum(mask, 1.0 - anym)


def _tc_any_mask(npos, bat, resp, G):
    N = npos.shape[0]
    R = 1000

    def body(np_ref, b_ref, r_ref, o_ref):
        i = pl.program_id(0)
        rsel = _dotg(_onehot(b_ref[...], G), r_ref[...])
        mask = (np_ref[...] < rsel).astype(F32)

        @pl.when(i == 0)
        def _():
            o_ref[...] = jnp.zeros_like(o_ref)

        o_ref[...] = jnp.maximum(o_ref[...],
                                 jnp.max(mask, axis=(0, 1), keepdims=True))

    return pl.pallas_call(
        body,
        grid=(N // R,),
        in_specs=[pl.BlockSpec((R, 1), lambda i: (i, 0)),
                  pl.BlockSpec((R, 1), lambda i: (i, 0)),
                  pl.BlockSpec((G, 1), lambda i: (0, 0))],
        out_specs=pl.BlockSpec((1, 1), lambda i: (0, 0)),
        out_shape=jax.ShapeDtypeStruct((1, 1), F32),
    )(npos, bat, resp)


def _tc_graph_key(h, npos, bat, resp, anym, wk, bk, G):
    N, H = h.shape
    R = 1000
    ngrid = N // R

    def body(h_ref, np_ref, b_ref, r_ref, a_ref, wk_ref, bk_ref, k_ref,
             hsum, cnts):
        i = pl.program_id(0)

        @pl.when(i == 0)
        def _():
            hsum[...] = jnp.zeros_like(hsum)
            cnts[...] = jnp.zeros_like(cnts)

        oneh = _onehot(b_ref[...], G)
        sel = _sel_mask(np_ref[...], b_ref[...], r_ref[...], a_ref[...], G)
        dn = (((0,), (0,)), ((), ()))
        cnts[...] += lax.dot_general(oneh, sel, dn,
                                     preferred_element_type=F32,
                                     precision=lax.Precision.HIGHEST)
        hsum[...] += lax.dot_general(oneh, h_ref[...] * sel, dn,
                                     preferred_element_type=F32,
                                     precision=lax.Precision.HIGHEST)

        @pl.when(i == ngrid - 1)
        def _():
            hm = hsum[...] / (cnts[...] + 1e-6)
            k_ref[...] = _dot(hm, wk_ref[...]) + bk_ref[...]

    return pl.pallas_call(
        body,
        grid=(ngrid,),
        in_specs=[pl.BlockSpec((R, H), lambda i: (i, 0)),
                  pl.BlockSpec((R, 1), lambda i: (i, 0)),
                  pl.BlockSpec((R, 1), lambda i: (i, 0)),
                  pl.BlockSpec((G, 1), lambda i: (0, 0)),
                  pl.BlockSpec((1, 1), lambda i: (0, 0)),
                  pl.BlockSpec((H, H), lambda i: (0, 0)),
                  pl.BlockSpec((1, H), lambda i: (0, 0))],
        out_specs=pl.BlockSpec((G, H), lambda i: (0, 0)),
        out_shape=jax.ShapeDtypeStruct((G, H), F32),
        scratch_shapes=[pltpu.VMEM((G, H), F32), pltpu.VMEM((G, 1), F32)],
    )(h, npos, bat, resp, anym, wk, bk.reshape(1, H))


def _tc_scores(h, npos, bat, resp, anym, k, wq, bq, G):
    N, H = h.shape
    R = 1000

    def body(h_ref, np_ref, b_ref, r_ref, a_ref, k_ref, wq_ref, bq_ref,
             sc_ref, smax_ref):
        i = pl.program_id(0)
        oneh = _onehot(b_ref[...], G)
        sel = _sel_mask(np_ref[...], b_ref[...], r_ref[...], a_ref[...], G)
        q = _dot(h_ref[...], wq_ref[...]) + bq_ref[...]
        kb = _dotg(oneh, k_ref[...])
        sc = jnp.sum(q * kb, axis=-1, keepdims=True)
        sc_ref[...] = sc
        masked = jnp.where(sel > 0, sc, -jnp.inf)

        @pl.when(i == 0)
        def _():
            smax_ref[...] = jnp.full_like(smax_ref, -jnp.inf)

        smax_ref[...] = jnp.maximum(
            smax_ref[...], jnp.max(masked, axis=(0, 1), keepdims=True))

    return pl.pallas_call(
        body,
        grid=(N // R,),
        in_specs=[pl.BlockSpec((R, H), lambda i: (i, 0)),
                  pl.BlockSpec((R, 1), lambda i: (i, 0)),
                  pl.BlockSpec((R, 1), lambda i: (i, 0)),
                  pl.BlockSpec((G, 1), lambda i: (0, 0)),
                  pl.BlockSpec((1, 1), lambda i: (0, 0)),
                  pl.BlockSpec((G, H), lambda i: (0, 0)),
                  pl.BlockSpec((H, H), lambda i: (0, 0)),
                  pl.BlockSpec((1, H), lambda i: (0, 0))],
        out_specs=[pl.BlockSpec((R, 1), lambda i: (i, 0)),
                   pl.BlockSpec((1, 1), lambda i: (0, 0))],
        out_shape=[jax.ShapeDtypeStruct((N, 1), F32),
                   jax.ShapeDtypeStruct((1, 1), F32)],
    )(h, npos, bat, resp, anym, k, wq, bq.reshape(1, H))


def _tc_denom(scores, npos, bat, resp, anym, smax, G):
    N = scores.shape[0]

    def body(sc_ref, np_ref, b_ref, r_ref, a_ref, sm_ref, o_ref):
        oneh = _onehot(b_ref[...], G)
        sel = _sel_mask(np_ref[...], b_ref[...], r_ref[...], a_ref[...], G)
        exp_s = jnp.where(sel > 0, jnp.exp(sc_ref[...] - sm_ref[...]), 0.0)
        dn = (((0,), (0,)), ((), ()))
        o_ref[...] = lax.dot_general(oneh, exp_s, dn,
                                     preferred_element_type=F32,
                                     precision=lax.Precision.HIGHEST) + 1e-8

    return pl.pallas_call(
        body,
        grid=(1,),
        in_specs=[pl.BlockSpec((N, 1), lambda i: (0, 0)),
                  pl.BlockSpec((N, 1), lambda i: (0, 0)),
                  pl.BlockSpec((N, 1), lambda i: (0, 0)),
                  pl.BlockSpec((G, 1), lambda i: (0, 0)),
                  pl.BlockSpec((1, 1), lambda i: (0, 0)),
                  pl.BlockSpec((1, 1), lambda i: (0, 0))],
        out_specs=pl.BlockSpec((G, 1), lambda i: (0, 0)),
        out_shape=jax.ShapeDtypeStruct((G, 1), F32),
    )(scores, npos, bat, resp, anym, smax)


def _tc_pool_logits(h, scores, npos, bat, resp, anym, smax, denom,
                    wc1, bc1, wc2, bc2, G):
    N, H = h.shape
    R = 1000
    ngrid = N // R
    Hc = wc1.shape[1]

    def body(h_ref, sc_ref, np_ref, b_ref, r_ref, a_ref, sm_ref, d_ref,
             wc1_ref, bc1_ref, wc2_ref, bc2_ref, o_ref, hg):
        i = pl.program_id(0)

        @pl.when(i == 0)
        def _():
            hg[...] = jnp.zeros_like(hg)

        oneh = _onehot(b_ref[...], G)
        sel = _sel_mask(np_ref[...], b_ref[...], r_ref[...], a_ref[...], G)
        exp_s = jnp.where(sel > 0, jnp.exp(sc_ref[...] - sm_ref[...]), 0.0)
        att = exp_s / _dotg(oneh, d_ref[...])
        dn = (((0,), (0,)), ((), ()))
        hg[...] += lax.dot_general(oneh, h_ref[...] * att, dn,
                                   preferred_element_type=F32,
                                   precision=lax.Precision.HIGHEST)

        @pl.when(i == ngrid - 1)
        def _():
            hc = jax.nn.relu(_dot(hg[...], wc1_ref[...]) + bc1_ref[...])
            o_ref[...] = _dot(hc, wc2_ref[...]) + bc2_ref[...]

    return pl.pallas_call(
        body,
        grid=(ngrid,),
        in_specs=[pl.BlockSpec((R, H), lambda i: (i, 0)),
                  pl.BlockSpec((R, 1), lambda i: (i, 0)),
                  pl.BlockSpec((R, 1), lambda i: (i, 0)),
                  pl.BlockSpec((R, 1), lambda i: (i, 0)),
                  pl.BlockSpec((G, 1), lambda i: (0, 0)),
                  pl.BlockSpec((1, 1), lambda i: (0, 0)),
                  pl.BlockSpec((1, 1), lambda i: (0, 0)),
                  pl.BlockSpec((G, 1), lambda i: (0, 0)),
                  pl.BlockSpec((H, Hc), lambda i: (0, 0)),
                  pl.BlockSpec((1, Hc), lambda i: (0, 0)),
                  pl.BlockSpec((Hc, 1), lambda i: (0, 0)),
                  pl.BlockSpec((1, 1), lambda i: (0, 0))],
        out_specs=pl.BlockSpec((G, 1), lambda i: (0, 0)),
        out_shape=jax.ShapeDtypeStruct((G, 1), F32),
        scratch_shapes=[pltpu.VMEM((G, H), F32)],
    )(h, scores, npos, bat, resp, anym, smax, denom,
      wc1, bc1.reshape(1, Hc), wc2, bc2.reshape(1, 1))


def kernel(x, he_index, he_attr, he_count, node_pos, response_idx, batch,
           params):
    N = x.shape[0]
    E, Dhe = he_attr.shape
    G = response_idx.shape[0]
    H = params['W_in'].shape[1]
    M = he_index.shape[1]

    node_ids = he_index[0]
    he_ids = he_index[1]

    mr = -(-(-(-M // 128)) // 256) * 256
    mp = mr * 128
    g_node0 = _pad_idx(node_ids, mp, 0)
    g_node1 = g_node0 + N
    g_he0 = _pad_idx(he_ids, mp, 0)
    g_he1 = g_he0 + E
    s_node = _pad_idx(node_ids, mp, N)
    s_he = _pad_idx(he_ids, mp, E)

    deg2 = _sc_scatter_add(None, None, None, s_node, N, H // 2,
                           split_by_core=True
                           ).reshape(2, N, H // 2)[:, :, 0:1]

    tok = (deg2[0, 0, 0] * 0.0).astype(jnp.int32)
    g_node0 = g_node0 + tok
    g_node1 = g_node1 + tok

    h = _tc_input_proj(x, params['W_in'], params['b_in'])
    cnt = he_count.reshape(E, 1)

    for lp in params['layers']:
        m2 = _tc_n2e(h, lp['n2e'])
        agg2 = _sc_scatter_add(m2.reshape(2 * N, H // 2), g_node0, g_node1,
                               s_he, E, H // 2,
                               split_by_core=False).reshape(2, E, H // 2)
        inc2 = _tc_e2n(he_attr, agg2, cnt, lp['e2n'])
        out2 = _sc_scatter_add(inc2.reshape(2 * E, H // 2), g_he0, g_he1,
                               s_node, N, H // 2,
                               split_by_core=False).reshape(2, N, H // 2)
        h = _tc_post(h, out2, deg2, lp['ln_g'], lp['ln_b'])

    npos = node_pos.astype(F32).reshape(N, 1)
    bat = batch.reshape(N, 1)
    resp = response_idx.astype(F32).reshape(G, 1)

    anym = _tc_any_mask(npos, bat, resp, G)
    k = _tc_graph_key(h, npos, bat, resp, anym, params['Wk'], params['bk'], G)
    scores, smax = _tc_scores(h, npos, bat, resp, anym, k,
                              params['Wq'], params['bq'], G)
    denom = _tc_denom(scores, npos, bat, resp, anym, smax, G)
    logits = _tc_pool_logits(h, scores, npos, bat, resp, anym, smax, denom,
                             params['Wc1'], params['bc1'],
                             params['Wc2'], params['bc2'], G)
    return logits.reshape(-1)
